# baseline (speedup 1.0000x reference)
import functools

import jax
import jax.numpy as jnp
from jax import lax
from jax.experimental import pallas as pl
from jax.experimental.pallas import tpu as pltpu
from jax.experimental.pallas import tpu_sc as plsc

NC = 2
NS = 16
NW = NC * NS
LANES = 16

CHUNK = 128
NBUF = 4
LOOKAHEAD = 3


def _make_kernel(N, V, L, H):
    assert N % NW == 0
    R = N // NW
    assert R % CHUNK == 0
    NCH = R // CHUNK
    assert NCH % NBUF == 0
    G = NCH // NBUF
    assert (R % L == 0) and (CHUNK % 8 == 0) and (L % 8 == 0)
    assert H % LANES == 0

    mesh = plsc.VectorSubcoreMesh(
        core_axis_name="c", subcore_axis_name="s", num_cores=NC,
        num_subcores=NS)

    @functools.partial(
        pl.kernel,
        out_type=jax.ShapeDtypeStruct((N, H), jnp.float32),
        mesh=mesh,
        scratch_types=dict(
            idx_all=pltpu.VMEM((R,), jnp.int32),
            pos2=pltpu.VMEM((2 * L, H), jnp.float32),
            rows=[pltpu.VMEM((CHUNK, H), jnp.float32) for _ in range(NBUF)],
            gsem=[pltpu.SemaphoreType.DMA for _ in range(NBUF)],
            ssem=[pltpu.SemaphoreType.DMA for _ in range(NBUF)],
        ),
        compiler_params=pltpu.CompilerParams(use_tc_tiling_on_sc=False),
    )
    def emb_kernel(x_hbm, emb_hbm, pos_hbm, out_hbm, *, idx_all, pos2, rows,
                   gsem, ssem):
        wid = lax.axis_index("s") * NC + lax.axis_index("c")
        base = wid * R

        pltpu.sync_copy(x_hbm.at[pl.ds(base, R)], idx_all)
        pltpu.sync_copy(pos_hbm, pos2.at[pl.ds(0, L)])
        pltpu.sync_copy(pos_hbm, pos2.at[pl.ds(L, L)])

        def gather_start(c, b):
            pltpu.async_copy(
                emb_hbm.at[idx_all.at[pl.ds(c * CHUNK, CHUNK)]], rows[b],
                gsem[b])

        def gather_wait(c, b):
            pltpu.make_async_copy(
                emb_hbm.at[idx_all.at[pl.ds(c * CHUNK, CHUNK)]], rows[b],
                gsem[b]).wait()

        def store_start(c, b):
            pltpu.async_copy(
                rows[b], out_hbm.at[pl.ds(base + c * CHUNK, CHUNK)], ssem[b])

        def store_wait(c, b):
            pltpu.make_async_copy(
                rows[b], out_hbm.at[pl.ds(base + c * CHUNK, CHUNK)],
                ssem[b]).wait()

        def add_pos(c, b):
            p0 = lax.rem(c * CHUNK, L)

            def rbody(r, carry):
                pr = p0 + r
                for j in range(H // LANES):
                    sl = pl.ds(j * LANES, LANES)
                    rows[b][r, sl] = rows[b][r, sl] + pos2[pr, sl]
                return carry

            lax.fori_loop(0, CHUNK, rbody, 0, unroll=2)

        for b in range(LOOKAHEAD):
            gather_start(b, b)

        def group(g, carry):
            for b in range(NBUF):
                c = g * NBUF + b
                gq = c + LOOKAHEAD
                bg = (b + LOOKAHEAD) % NBUF

                @pl.when(gq < NCH)
                def _issue():
                    @pl.when(gq >= NBUF)
                    def _wait_store():
                        store_wait(gq - NBUF, bg)

                    gather_start(gq, bg)

                gather_wait(c, b)
                add_pos(c, b)
                store_start(c, b)
            return carry

        lax.fori_loop(0, G, group, 0)

        for b in range(NBUF):
            store_wait(NCH - NBUF + b, b)

    return emb_kernel


def kernel(x, emb_table, pos_table):
    B, L = x.shape
    V, H = emb_table.shape
    N = B * L
    x_flat = x.reshape(N).astype(jnp.int32)
    fn = _make_kernel(N, V, L, H)
    out = fn(x_flat, emb_table, pos_table)
    return out.reshape(B, L, H)
